# baseline (speedup 1.0000x reference)
import functools

import jax
import jax.numpy as jnp
from jax import lax
from jax.experimental import pallas as pl
from jax.experimental.pallas import tpu as pltpu
from jax.experimental.pallas import tpu_sc as plsc

N = 10000
E = 160000
D_IN = 256
D_H = 512
NUM_GRAPHS = 8
NUM_EXPERTS = 8

NR = 10240
EB = 128
EP = 163840
ROWS_ALL = EP // EB
ROWS_TILE = ROWS_ALL // 16
ROWS_TILE32 = ROWS_ALL // 32
ROWS_GRP = 16
RPT = NR // 16
ET32 = EP // 32
MR = (NUM_GRAPHS + 1) * NR
MRPT = MR // 16
BR = 1024
NBLK = NR // BR

_mesh = plsc.VectorSubcoreMesh(core_axis_name="c", subcore_axis_name="s")


@functools.partial(
    pl.kernel,
    out_type=jax.ShapeDtypeStruct((2, NR), jnp.float32),
    mesh=_mesh,
    scratch_types=[
        pltpu.VMEM((ROWS_TILE32, EB), jnp.int32),
        pltpu.VMEM((RPT,), jnp.float32),
        pltpu.VMEM((EB,), jnp.float32),
        pltpu.VMEM_SHARED((NR,), jnp.float32),
    ],
)
def _sc_deg(dst_hbm, out_hbm, idx_d, zbuf, ones, acc):
    c = lax.axis_index("c")
    s = lax.axis_index("s")
    w = c * 16 + s
    pltpu.sync_copy(dst_hbm.at[pl.ds(w * ROWS_TILE32, ROWS_TILE32)], idx_d)

    def _fill(i, _):
        zbuf[pl.ds(i * 16, 16)] = jnp.zeros((16,), jnp.float32)
        return 0
    lax.fori_loop(0, RPT // 16, _fill, 0)

    def _fill1(i, _):
        ones[pl.ds(i * 16, 16)] = jnp.full((16,), 1.0, jnp.float32)
        return 0
    lax.fori_loop(0, EB // 16, _fill1, 0)

    pltpu.sync_copy(zbuf, acc.at[pl.ds(s * RPT, RPT)])
    plsc.subcore_barrier()

    def _scat(j, _):
        pltpu.sync_copy(ones, acc.at[idx_d.at[j]], add=True)
        return 0
    lax.fori_loop(0, ROWS_TILE32, _scat, 0)
    plsc.subcore_barrier()
    pltpu.sync_copy(acc.at[pl.ds(s * RPT, RPT)], out_hbm.at[c, pl.ds(s * RPT, RPT)])


def _make_sc_agg(nch):
    per = nch // 2

    @functools.partial(
        pl.kernel,
        out_type=jax.ShapeDtypeStruct((nch, NR, 128), jnp.float32),
        mesh=_mesh,
        scratch_types=[
            pltpu.VMEM((ROWS_GRP, EB), jnp.int32),
            pltpu.VMEM((ROWS_GRP, EB), jnp.int32),
            pltpu.VMEM((EB, 128), jnp.float32),
            pltpu.VMEM((EB, 128), jnp.float32),
            pltpu.SemaphoreType.DMA,
            pltpu.SemaphoreType.DMA,
            pltpu.VMEM_SHARED((NR, 128), jnp.float32),
        ],
    )
    def _agg(t_hbm, src_hbm, dst_hbm, out_hbm, idx_s, idx_d, bufa, bufb,
             sema, semb, acc):
        c = lax.axis_index("c")
        s = lax.axis_index("s")
        base = s * ROWS_TILE
        row0 = s * RPT
        for k in range(per):
            chunk = per * c + k
            tbl = t_hbm.at[chunk]
            pltpu.sync_copy(tbl.at[pl.ds(row0, RPT)], acc.at[pl.ds(row0, RPT)])
            plsc.subcore_barrier()

            for g in range(ROWS_TILE // ROWS_GRP):
                pltpu.sync_copy(
                    src_hbm.at[pl.ds(base + g * ROWS_GRP, ROWS_GRP)], idx_s)
                pltpu.sync_copy(
                    dst_hbm.at[pl.ds(base + g * ROWS_GRP, ROWS_GRP)], idx_d)
                pltpu.async_copy(tbl.at[idx_s.at[0]], bufa, sema)

                def _pair(i, _):
                    j0 = 2 * i
                    pltpu.async_copy(tbl.at[idx_s.at[j0 + 1]], bufb, semb)
                    pltpu.make_async_copy(tbl.at[idx_s.at[j0]], bufa,
                                          sema).wait()
                    pltpu.sync_copy(bufa, acc.at[idx_d.at[j0]], add=True)

                    @pl.when(i < ROWS_GRP // 2 - 1)
                    def _():
                        pltpu.async_copy(tbl.at[idx_s.at[j0 + 2]], bufa, sema)

                    pltpu.make_async_copy(tbl.at[idx_s.at[j0 + 1]], bufb,
                                          semb).wait()
                    pltpu.sync_copy(bufb, acc.at[idx_d.at[j0 + 1]], add=True)
                    return 0
                lax.fori_loop(0, ROWS_GRP // 2, _pair, 0)
            plsc.subcore_barrier()
            pltpu.sync_copy(acc.at[pl.ds(row0, RPT)],
                            out_hbm.at[chunk, pl.ds(row0, RPT)])
            plsc.subcore_barrier()
    return _agg


_sc_agg2 = _make_sc_agg(2)
_sc_agg4 = _make_sc_agg(4)


@functools.partial(
    pl.kernel,
    out_type=jax.ShapeDtypeStruct((2, MR), jnp.float32),
    mesh=_mesh,
    scratch_types=[
        pltpu.VMEM((ET32,), jnp.int32),
        pltpu.VMEM((ET32,), jnp.int32),
        pltpu.VMEM((NR,), jnp.int32),
        pltpu.VMEM((NR,), jnp.float32),
        pltpu.VMEM((EB,), jnp.int32),
        pltpu.VMEM((EB,), jnp.float32),
        pltpu.VMEM((MRPT,), jnp.float32),
        pltpu.VMEM_SHARED((MR,), jnp.float32),
    ],
    compiler_params=pltpu.CompilerParams(needs_layout_passes=False),
)
def _sc_m(src_hbm, dst_hbm, batch_hbm, dinv_hbm, out_hbm,
          ssrc, sdst, btab, dtab, fx, fv, zbuf, accm):
    c = lax.axis_index("c")
    s = lax.axis_index("s")
    w = c * 16 + s
    pltpu.sync_copy(src_hbm.at[pl.ds(w * ET32, ET32)], ssrc)
    pltpu.sync_copy(dst_hbm.at[pl.ds(w * ET32, ET32)], sdst)
    pltpu.sync_copy(batch_hbm, btab)
    pltpu.sync_copy(dinv_hbm, dtab)

    def _fill(i, _):
        zbuf[pl.ds(i * 16, 16)] = jnp.zeros((16,), jnp.float32)
        return 0
    lax.fori_loop(0, MRPT // 16, _fill, 0)
    pltpu.sync_copy(zbuf, accm.at[pl.ds(s * MRPT, MRPT)])
    plsc.subcore_barrier()

    def _blk(j, _):
        for m in range(EB // 16):
            off = j * EB + m * 16
            d16 = sdst[pl.ds(off, 16)]
            s16 = ssrc[pl.ds(off, 16)]
            b16 = plsc.load_gather(btab, [d16])
            v16 = plsc.load_gather(dtab, [d16])
            fx[pl.ds(m * 16, 16)] = b16 * NR + s16
            fv[pl.ds(m * 16, 16)] = v16
        pltpu.sync_copy(fv, accm.at[fx], add=True)
        return 0
    lax.fori_loop(0, ET32 // EB, _blk, 0)
    plsc.subcore_barrier()
    pltpu.sync_copy(accm.at[pl.ds(s * MRPT, MRPT)],
                    out_hbm.at[c, pl.ds(s * MRPT, MRPT)])


@functools.partial(
    pl.kernel,
    out_type=jax.ShapeDtypeStruct((2, NR, 128), jnp.float32),
    mesh=_mesh,
    scratch_types=[
        pltpu.VMEM((ROWS_GRP, EB), jnp.int32),
        pltpu.VMEM((EB, 256), jnp.float32),
        pltpu.VMEM((EB, 256), jnp.float32),
        pltpu.SemaphoreType.DMA,
        pltpu.SemaphoreType.DMA,
        pltpu.VMEM_SHARED((NR // 2, 128), jnp.float32),
    ],
)
def _sc_probe(t_hbm, src_hbm, dst_hbm, out_hbm, idx_s, bufa, bufb,
              sema, semb, acc):
    c = lax.axis_index("c")
    s = lax.axis_index("s")
    base = s * ROWS_TILE
    row0 = s * RPT
    for g in range(ROWS_TILE // ROWS_GRP):
        pltpu.sync_copy(src_hbm.at[pl.ds(base + g * ROWS_GRP, ROWS_GRP)],
                        idx_s)
        pltpu.async_copy(t_hbm.at[idx_s.at[0]], bufa, sema)

        def _pair(i, _):
            j0 = 2 * i
            pltpu.async_copy(t_hbm.at[idx_s.at[j0 + 1]], bufb, semb)
            pltpu.make_async_copy(t_hbm.at[idx_s.at[j0]], bufa, sema).wait()

            @pl.when(i < ROWS_GRP // 2 - 1)
            def _():
                pltpu.async_copy(t_hbm.at[idx_s.at[j0 + 2]], bufa, sema)

            pltpu.make_async_copy(t_hbm.at[idx_s.at[j0 + 1]], bufb,
                                  semb).wait()
            return 0
        lax.fori_loop(0, ROWS_GRP // 2, _pair, 0)
    plsc.subcore_barrier()
    pltpu.sync_copy(acc.at[pl.ds(s * 320, 320)],
                    out_hbm.at[0, pl.ds(s * 320, 320)])


def _dinv_of(deg_ref):
    cnt = deg_ref[0, :] + deg_ref[1, :]
    return lax.rsqrt(cnt + 1.0)


def _tc_dinv(deg2):
    def body(deg_ref, out_ref):
        out_ref[0, 0] = _dinv_of(deg_ref)

    return pl.pallas_call(
        body,
        grid=(NBLK,),
        in_specs=[pl.BlockSpec((2, BR), lambda i: (0, i))],
        out_specs=pl.BlockSpec((1, 1, BR), lambda i: (i, 0, 0)),
        out_shape=jax.ShapeDtypeStruct((NBLK, 1, BR), jnp.float32),
    )(deg2)


def _tc_scale_x(xp, deg2):
    def body(x_ref, deg_ref, out_ref):
        xs = x_ref[...] * _dinv_of(deg_ref)[:, None]
        for cc in range(2):
            out_ref[cc] = xs[:, cc * 128:(cc + 1) * 128]

    return pl.pallas_call(
        body,
        grid=(NBLK,),
        in_specs=[
            pl.BlockSpec((BR, D_IN), lambda i: (i, 0)),
            pl.BlockSpec((2, BR), lambda i: (0, i)),
        ],
        out_specs=pl.BlockSpec((2, BR, 128), lambda i: (0, i, 0)),
        out_shape=jax.ShapeDtypeStruct((2, NR, 128), jnp.float32),
    )(xp, deg2)


def _tc_l1(Sx, deg2, W1, b1, W2):
    def body(s_ref, deg_ref, w1_ref, b1_ref, w2_ref, out_ref):
        ax = jnp.concatenate([s_ref[0], s_ref[1]], axis=1)
        dinv = _dinv_of(deg_ref)
        ax = ax * dinv[:, None]
        h1 = jnp.dot(ax, w1_ref[...], preferred_element_type=jnp.float32)
        u = jnp.maximum(h1 + b1_ref[...], 0.0)
        t = jnp.dot(u, w2_ref[...], preferred_element_type=jnp.float32)
        t = t * dinv[:, None]
        for cc in range(4):
            out_ref[cc] = t[:, cc * 128:(cc + 1) * 128]

    return pl.pallas_call(
        body,
        grid=(NBLK,),
        in_specs=[
            pl.BlockSpec((2, BR, 128), lambda i: (0, i, 0)),
            pl.BlockSpec((2, BR), lambda i: (0, i)),
            pl.BlockSpec((D_IN, D_H), lambda i: (0, 0)),
            pl.BlockSpec((1, D_H), lambda i: (0, 0)),
            pl.BlockSpec((D_H, D_H), lambda i: (0, 0)),
        ],
        out_specs=pl.BlockSpec((4, BR, 128), lambda i: (0, i, 0)),
        out_shape=jax.ShapeDtypeStruct((4, NR, 128), jnp.float32),
    )(Sx, deg2, W1, b1, W2)


def _tc_l3pool(S2, deg2, b2, batch2d, Mp3, W3, b3, Wm1, bm1, Wm2, bm2):
    def body(s_ref, deg_ref, b2_ref, batch_ref, mp_ref, w3_ref, b3_ref,
             wm1_ref, bm1_ref, wm2_ref, bm2_ref, out_ref, musum, cnts):
        i = pl.program_id(0)

        @pl.when(i == 0)
        def _init():
            musum[...] = jnp.zeros((NUM_GRAPHS, D_H), jnp.float32)
            cnts[...] = jnp.zeros((NUM_GRAPHS, 128), jnp.float32)

        h = jnp.concatenate([s_ref[cc] for cc in range(4)], axis=1)
        dinv = _dinv_of(deg_ref)
        u3 = jnp.maximum(h * dinv[:, None] + b2_ref[...], 0.0)
        bvec = batch_ref[0, 0]
        gids = lax.broadcasted_iota(jnp.int32, (NUM_GRAPHS, 1), 0)
        pg = (bvec[None, :] == gids).astype(jnp.float32)
        mblk = mp_ref[0, :NUM_GRAPHS, :] + mp_ref[1, :NUM_GRAPHS, :]
        mfull = (mblk + pg * dinv[None, :]) * dinv[None, :]
        dn = (((1,), (0,)), ((), ()))
        musum[...] += lax.dot_general(mfull, u3, dn,
                                      preferred_element_type=jnp.float32)
        cnts[...] += lax.dot_general(pg, jnp.ones((BR, 128), jnp.float32), dn,
                                     preferred_element_type=jnp.float32)

        @pl.when(i == NBLK - 1)
        def _fin():
            cnt = cnts[:, 0:1]
            zsum = jnp.dot(musum[...], w3_ref[...],
                           preferred_element_type=jnp.float32) + cnt * b3_ref[...]
            z = zsum / jnp.maximum(cnt, 1.0)
            z = jnp.maximum(
                jnp.dot(z, wm1_ref[...], preferred_element_type=jnp.float32)
                + bm1_ref[...], 0.0)
            out_ref[...] = (
                jnp.dot(z, wm2_ref[...], preferred_element_type=jnp.float32)
                + bm2_ref[...])

    return pl.pallas_call(
        body,
        grid=(NBLK,),
        in_specs=[
            pl.BlockSpec((4, BR, 128), lambda i: (0, i, 0)),
            pl.BlockSpec((2, BR), lambda i: (0, i)),
            pl.BlockSpec((1, D_H), lambda i: (0, 0)),
            pl.BlockSpec((1, 1, BR), lambda i: (i, 0, 0)),
            pl.BlockSpec((2, NUM_GRAPHS + 1, BR), lambda i: (0, 0, i)),
            pl.BlockSpec((D_H, D_H), lambda i: (0, 0)),
            pl.BlockSpec((1, D_H), lambda i: (0, 0)),
            pl.BlockSpec((D_H, D_H // 2), lambda i: (0, 0)),
            pl.BlockSpec((1, D_H // 2), lambda i: (0, 0)),
            pl.BlockSpec((D_H // 2, NUM_EXPERTS), lambda i: (0, 0)),
            pl.BlockSpec((1, NUM_EXPERTS), lambda i: (0, 0)),
        ],
        out_specs=pl.BlockSpec((NUM_GRAPHS, NUM_EXPERTS), lambda i: (0, 0)),
        out_shape=jax.ShapeDtypeStruct((NUM_GRAPHS, NUM_EXPERTS), jnp.float32),
        scratch_shapes=[
            pltpu.VMEM((NUM_GRAPHS, D_H), jnp.float32),
            pltpu.VMEM((NUM_GRAPHS, 128), jnp.float32),
        ],
    )(S2, deg2, b2, batch2d, Mp3, W3, b3, Wm1, bm1, Wm2, bm2)


def kernel(x, edge_index, batch, W1, b1, W2, b2, W3, b3, Wm1, bm1, Wm2, bm2):
    src = edge_index[0]
    dst = edge_index[1]
    src2d = jnp.pad(src, (0, EP - E)).reshape(ROWS_ALL, EB)
    dst2d = jnp.pad(dst, (0, EP - E), constant_values=N).reshape(ROWS_ALL, EB)
    xp = jnp.pad(x, ((0, NR - N), (0, 0)))
    batch2d = jnp.pad(batch, (0, NR - N),
                      constant_values=NUM_GRAPHS).reshape(NBLK, 1, BR)

    deg2 = _sc_deg(dst2d)
    dinv_flat = _tc_dinv(deg2).reshape(NR)
    batch_flat = jnp.pad(batch, (0, NR - N), constant_values=NUM_GRAPHS)
    Mp = _sc_m(src2d.reshape(EP), dst2d.reshape(EP), batch_flat, dinv_flat)
    xs = _tc_scale_x(xp, deg2)
    Sx = _sc_probe(jnp.reshape(xs, (NR, 256)), src2d, dst2d)
    t2 = _tc_l1(Sx, deg2, W1, b1.reshape(1, D_H), W2)
    S2 = _sc_agg4(t2, src2d, dst2d)
    return _tc_l3pool(S2, deg2, b2.reshape(1, D_H), batch2d,
                      Mp.reshape(2, NUM_GRAPHS + 1, NR), W3,
                      b3.reshape(1, D_H), Wm1, bm1.reshape(1, D_H // 2),
                      Wm2, bm2.reshape(1, NUM_EXPERTS))


# speedup vs baseline: 1.1158x; 1.1158x over previous
import functools

import jax
import jax.numpy as jnp
from jax import lax
from jax.experimental import pallas as pl
from jax.experimental.pallas import tpu as pltpu
from jax.experimental.pallas import tpu_sc as plsc

N = 10000
E = 160000
D_IN = 256
D_H = 512
NUM_GRAPHS = 8
NUM_EXPERTS = 8

NR = 10240
EB = 128
EP = 163840
ROWS_ALL = EP // EB
ROWS_TILE = ROWS_ALL // 16
ROWS_TILE32 = ROWS_ALL // 32
ROWS_GRP = 16
RPT = NR // 16
ET32 = EP // 32
MR = (NUM_GRAPHS + 1) * NR
MRPT = MR // 16
BR = 1024
NBLK = NR // BR

_mesh = plsc.VectorSubcoreMesh(core_axis_name="c", subcore_axis_name="s")


@functools.partial(
    pl.kernel,
    out_type=jax.ShapeDtypeStruct((2, NR), jnp.float32),
    mesh=_mesh,
    scratch_types=[
        pltpu.VMEM((ROWS_TILE32, EB), jnp.int32),
        pltpu.VMEM((RPT,), jnp.float32),
        pltpu.VMEM((EB,), jnp.float32),
        pltpu.VMEM_SHARED((NR,), jnp.float32),
    ],
)
def _sc_deg(dst_hbm, out_hbm, idx_d, zbuf, ones, acc):
    c = lax.axis_index("c")
    s = lax.axis_index("s")
    w = c * 16 + s
    pltpu.sync_copy(dst_hbm.at[pl.ds(w * ROWS_TILE32, ROWS_TILE32)], idx_d)

    def _fill(i, _):
        zbuf[pl.ds(i * 16, 16)] = jnp.zeros((16,), jnp.float32)
        return 0
    lax.fori_loop(0, RPT // 16, _fill, 0)

    def _fill1(i, _):
        ones[pl.ds(i * 16, 16)] = jnp.full((16,), 1.0, jnp.float32)
        return 0
    lax.fori_loop(0, EB // 16, _fill1, 0)

    pltpu.sync_copy(zbuf, acc.at[pl.ds(s * RPT, RPT)])
    plsc.subcore_barrier()

    def _scat(j, _):
        pltpu.sync_copy(ones, acc.at[idx_d.at[j]], add=True)
        return 0
    lax.fori_loop(0, ROWS_TILE32, _scat, 0)
    plsc.subcore_barrier()
    pltpu.sync_copy(acc.at[pl.ds(s * RPT, RPT)], out_hbm.at[c, pl.ds(s * RPT, RPT)])


def _make_sc_agg(nch):
    per = nch // 2

    @functools.partial(
        pl.kernel,
        out_type=jax.ShapeDtypeStruct((nch, NR, 128), jnp.float32),
        mesh=_mesh,
        scratch_types=[
            pltpu.VMEM((ROWS_GRP, EB), jnp.int32),
            pltpu.VMEM((ROWS_GRP, EB), jnp.int32),
            pltpu.VMEM((EB, 128), jnp.float32),
            pltpu.VMEM((EB, 128), jnp.float32),
            pltpu.SemaphoreType.DMA,
            pltpu.SemaphoreType.DMA,
            pltpu.VMEM_SHARED((NR, 128), jnp.float32),
        ],
    )
    def _agg(t_hbm, src_hbm, dst_hbm, out_hbm, idx_s, idx_d, bufa, bufb,
             sema, semb, acc):
        c = lax.axis_index("c")
        s = lax.axis_index("s")
        base = s * ROWS_TILE
        row0 = s * RPT
        for k in range(per):
            chunk = per * c + k
            tbl = t_hbm.at[chunk]
            pltpu.sync_copy(tbl.at[pl.ds(row0, RPT)], acc.at[pl.ds(row0, RPT)])
            plsc.subcore_barrier()

            for g in range(ROWS_TILE // ROWS_GRP):
                pltpu.sync_copy(
                    src_hbm.at[pl.ds(base + g * ROWS_GRP, ROWS_GRP)], idx_s)
                pltpu.sync_copy(
                    dst_hbm.at[pl.ds(base + g * ROWS_GRP, ROWS_GRP)], idx_d)
                pltpu.async_copy(tbl.at[idx_s.at[0, pl.ds(0, 64)]],
                                 bufa.at[pl.ds(0, 64)], sema)
                pltpu.async_copy(tbl.at[idx_s.at[0, pl.ds(64, 64)]],
                                 bufa.at[pl.ds(64, 64)], sema)

                def _fire(j, buf, sem):
                    pltpu.async_copy(tbl.at[idx_s.at[j, pl.ds(0, 64)]],
                                     buf.at[pl.ds(0, 64)], sem)
                    pltpu.async_copy(tbl.at[idx_s.at[j, pl.ds(64, 64)]],
                                     buf.at[pl.ds(64, 64)], sem)

                def _pair(i, _):
                    j0 = 2 * i
                    _fire(j0 + 1, bufb, semb)
                    pltpu.make_async_copy(tbl.at[idx_s.at[j0]], bufa,
                                          sema).wait()
                    pltpu.sync_copy(bufa, acc.at[idx_d.at[j0]], add=True)

                    @pl.when(i < ROWS_GRP // 2 - 1)
                    def _():
                        _fire(j0 + 2, bufa, sema)

                    pltpu.make_async_copy(tbl.at[idx_s.at[j0 + 1]], bufb,
                                          semb).wait()
                    pltpu.sync_copy(bufb, acc.at[idx_d.at[j0 + 1]], add=True)
                    return 0
                lax.fori_loop(0, ROWS_GRP // 2, _pair, 0)
            plsc.subcore_barrier()
            pltpu.sync_copy(acc.at[pl.ds(row0, RPT)],
                            out_hbm.at[chunk, pl.ds(row0, RPT)])
            plsc.subcore_barrier()
    return _agg


_sc_agg2 = _make_sc_agg(2)
_sc_agg4 = _make_sc_agg(4)


@functools.partial(
    pl.kernel,
    out_type=jax.ShapeDtypeStruct((2, MR), jnp.float32),
    mesh=_mesh,
    scratch_types=[
        pltpu.VMEM((ET32,), jnp.int32),
        pltpu.VMEM((ET32,), jnp.int32),
        pltpu.VMEM((NR,), jnp.int32),
        pltpu.VMEM((NR,), jnp.float32),
        pltpu.VMEM((EB,), jnp.int32),
        pltpu.VMEM((EB,), jnp.float32),
        pltpu.VMEM((MRPT,), jnp.float32),
        pltpu.VMEM_SHARED((MR,), jnp.float32),
    ],
    compiler_params=pltpu.CompilerParams(needs_layout_passes=False),
)
def _sc_m(src_hbm, dst_hbm, batch_hbm, dinv_hbm, out_hbm,
          ssrc, sdst, btab, dtab, fx, fv, zbuf, accm):
    c = lax.axis_index("c")
    s = lax.axis_index("s")
    w = c * 16 + s
    pltpu.sync_copy(src_hbm.at[pl.ds(w * ET32, ET32)], ssrc)
    pltpu.sync_copy(dst_hbm.at[pl.ds(w * ET32, ET32)], sdst)
    pltpu.sync_copy(batch_hbm, btab)
    pltpu.sync_copy(dinv_hbm, dtab)

    def _fill(i, _):
        zbuf[pl.ds(i * 16, 16)] = jnp.zeros((16,), jnp.float32)
        return 0
    lax.fori_loop(0, MRPT // 16, _fill, 0)
    pltpu.sync_copy(zbuf, accm.at[pl.ds(s * MRPT, MRPT)])
    plsc.subcore_barrier()

    def _blk(j, _):
        for m in range(EB // 16):
            off = j * EB + m * 16
            d16 = sdst[pl.ds(off, 16)]
            s16 = ssrc[pl.ds(off, 16)]
            b16 = plsc.load_gather(btab, [d16])
            v16 = plsc.load_gather(dtab, [d16])
            fx[pl.ds(m * 16, 16)] = b16 * NR + s16
            fv[pl.ds(m * 16, 16)] = v16
        pltpu.sync_copy(fv, accm.at[fx], add=True)
        return 0
    lax.fori_loop(0, ET32 // EB, _blk, 0)
    plsc.subcore_barrier()
    pltpu.sync_copy(accm.at[pl.ds(s * MRPT, MRPT)],
                    out_hbm.at[c, pl.ds(s * MRPT, MRPT)])


def _dinv_of(deg_ref):
    cnt = deg_ref[0, :] + deg_ref[1, :]
    return lax.rsqrt(cnt + 1.0)


def _tc_dinv(deg2):
    def body(deg_ref, out_ref):
        out_ref[0, 0] = _dinv_of(deg_ref)

    return pl.pallas_call(
        body,
        grid=(NBLK,),
        in_specs=[pl.BlockSpec((2, BR), lambda i: (0, i))],
        out_specs=pl.BlockSpec((1, 1, BR), lambda i: (i, 0, 0)),
        out_shape=jax.ShapeDtypeStruct((NBLK, 1, BR), jnp.float32),
    )(deg2)


def _tc_scale_x(xp, deg2):
    def body(x_ref, deg_ref, out_ref):
        xs = x_ref[...] * _dinv_of(deg_ref)[:, None]
        for cc in range(2):
            out_ref[cc] = xs[:, cc * 128:(cc + 1) * 128]

    return pl.pallas_call(
        body,
        grid=(NBLK,),
        in_specs=[
            pl.BlockSpec((BR, D_IN), lambda i: (i, 0)),
            pl.BlockSpec((2, BR), lambda i: (0, i)),
        ],
        out_specs=pl.BlockSpec((2, BR, 128), lambda i: (0, i, 0)),
        out_shape=jax.ShapeDtypeStruct((2, NR, 128), jnp.float32),
    )(xp, deg2)


def _tc_l1(Sx, deg2, W1, b1, W2):
    def body(s_ref, deg_ref, w1_ref, b1_ref, w2_ref, out_ref):
        ax = jnp.concatenate([s_ref[0], s_ref[1]], axis=1)
        dinv = _dinv_of(deg_ref)
        ax = ax * dinv[:, None]
        h1 = jnp.dot(ax, w1_ref[...], preferred_element_type=jnp.float32)
        u = jnp.maximum(h1 + b1_ref[...], 0.0)
        t = jnp.dot(u, w2_ref[...], preferred_element_type=jnp.float32)
        t = t * dinv[:, None]
        for cc in range(4):
            out_ref[cc] = t[:, cc * 128:(cc + 1) * 128]

    return pl.pallas_call(
        body,
        grid=(NBLK,),
        in_specs=[
            pl.BlockSpec((2, BR, 128), lambda i: (0, i, 0)),
            pl.BlockSpec((2, BR), lambda i: (0, i)),
            pl.BlockSpec((D_IN, D_H), lambda i: (0, 0)),
            pl.BlockSpec((1, D_H), lambda i: (0, 0)),
            pl.BlockSpec((D_H, D_H), lambda i: (0, 0)),
        ],
        out_specs=pl.BlockSpec((4, BR, 128), lambda i: (0, i, 0)),
        out_shape=jax.ShapeDtypeStruct((4, NR, 128), jnp.float32),
    )(Sx, deg2, W1, b1, W2)


def _tc_l3pool(S2, deg2, b2, batch2d, Mp3, W3, b3, Wm1, bm1, Wm2, bm2):
    def body(s_ref, deg_ref, b2_ref, batch_ref, mp_ref, w3_ref, b3_ref,
             wm1_ref, bm1_ref, wm2_ref, bm2_ref, out_ref, musum, cnts):
        i = pl.program_id(0)

        @pl.when(i == 0)
        def _init():
            musum[...] = jnp.zeros((NUM_GRAPHS, D_H), jnp.float32)
            cnts[...] = jnp.zeros((NUM_GRAPHS, 128), jnp.float32)

        h = jnp.concatenate([s_ref[cc] for cc in range(4)], axis=1)
        dinv = _dinv_of(deg_ref)
        u3 = jnp.maximum(h * dinv[:, None] + b2_ref[...], 0.0)
        bvec = batch_ref[0, 0]
        gids = lax.broadcasted_iota(jnp.int32, (NUM_GRAPHS, 1), 0)
        pg = (bvec[None, :] == gids).astype(jnp.float32)
        mblk = mp_ref[0, :NUM_GRAPHS, :] + mp_ref[1, :NUM_GRAPHS, :]
        mfull = (mblk + pg * dinv[None, :]) * dinv[None, :]
        dn = (((1,), (0,)), ((), ()))
        musum[...] += lax.dot_general(mfull, u3, dn,
                                      preferred_element_type=jnp.float32)
        cnts[...] += lax.dot_general(pg, jnp.ones((BR, 128), jnp.float32), dn,
                                     preferred_element_type=jnp.float32)

        @pl.when(i == NBLK - 1)
        def _fin():
            cnt = cnts[:, 0:1]
            zsum = jnp.dot(musum[...], w3_ref[...],
                           preferred_element_type=jnp.float32) + cnt * b3_ref[...]
            z = zsum / jnp.maximum(cnt, 1.0)
            z = jnp.maximum(
                jnp.dot(z, wm1_ref[...], preferred_element_type=jnp.float32)
                + bm1_ref[...], 0.0)
            out_ref[...] = (
                jnp.dot(z, wm2_ref[...], preferred_element_type=jnp.float32)
                + bm2_ref[...])

    return pl.pallas_call(
        body,
        grid=(NBLK,),
        in_specs=[
            pl.BlockSpec((4, BR, 128), lambda i: (0, i, 0)),
            pl.BlockSpec((2, BR), lambda i: (0, i)),
            pl.BlockSpec((1, D_H), lambda i: (0, 0)),
            pl.BlockSpec((1, 1, BR), lambda i: (i, 0, 0)),
            pl.BlockSpec((2, NUM_GRAPHS + 1, BR), lambda i: (0, 0, i)),
            pl.BlockSpec((D_H, D_H), lambda i: (0, 0)),
            pl.BlockSpec((1, D_H), lambda i: (0, 0)),
            pl.BlockSpec((D_H, D_H // 2), lambda i: (0, 0)),
            pl.BlockSpec((1, D_H // 2), lambda i: (0, 0)),
            pl.BlockSpec((D_H // 2, NUM_EXPERTS), lambda i: (0, 0)),
            pl.BlockSpec((1, NUM_EXPERTS), lambda i: (0, 0)),
        ],
        out_specs=pl.BlockSpec((NUM_GRAPHS, NUM_EXPERTS), lambda i: (0, 0)),
        out_shape=jax.ShapeDtypeStruct((NUM_GRAPHS, NUM_EXPERTS), jnp.float32),
        scratch_shapes=[
            pltpu.VMEM((NUM_GRAPHS, D_H), jnp.float32),
            pltpu.VMEM((NUM_GRAPHS, 128), jnp.float32),
        ],
    )(S2, deg2, b2, batch2d, Mp3, W3, b3, Wm1, bm1, Wm2, bm2)


def kernel(x, edge_index, batch, W1, b1, W2, b2, W3, b3, Wm1, bm1, Wm2, bm2):
    src = edge_index[0]
    dst = edge_index[1]
    src2d = jnp.pad(src, (0, EP - E)).reshape(ROWS_ALL, EB)
    dst2d = jnp.pad(dst, (0, EP - E), constant_values=N).reshape(ROWS_ALL, EB)
    xp = jnp.pad(x, ((0, NR - N), (0, 0)))
    batch2d = jnp.pad(batch, (0, NR - N),
                      constant_values=NUM_GRAPHS).reshape(NBLK, 1, BR)

    deg2 = _sc_deg(dst2d)
    dinv_flat = _tc_dinv(deg2).reshape(NR)
    batch_flat = jnp.pad(batch, (0, NR - N), constant_values=NUM_GRAPHS)
    Mp = _sc_m(src2d.reshape(EP), dst2d.reshape(EP), batch_flat, dinv_flat)
    xs = _tc_scale_x(xp, deg2)
    Sx = _sc_agg2(xs, src2d, dst2d)
    t2 = _tc_l1(Sx, deg2, W1, b1.reshape(1, D_H), W2)
    S2 = _sc_agg4(t2, src2d, dst2d)
    return _tc_l3pool(S2, deg2, b2.reshape(1, D_H), batch2d,
                      Mp.reshape(2, NUM_GRAPHS + 1, NR), W3,
                      b3.reshape(1, D_H), Wm1, bm1.reshape(1, D_H // 2),
                      Wm2, bm2.reshape(1, NUM_EXPERTS))


# speedup vs baseline: 1.1186x; 1.0026x over previous
import functools

import jax
import jax.numpy as jnp
from jax import lax
from jax.experimental import pallas as pl
from jax.experimental.pallas import tpu as pltpu
from jax.experimental.pallas import tpu_sc as plsc

N = 10000
E = 160000
D_IN = 256
D_H = 512
NUM_GRAPHS = 8
NUM_EXPERTS = 8

NR = 10240
EB = 128
EP = 163840
ROWS_ALL = EP // EB
ROWS_TILE = ROWS_ALL // 16
ROWS_TILE32 = ROWS_ALL // 32
ROWS_GRP = 16
RPT = NR // 16
ET32 = EP // 32
MR = (NUM_GRAPHS + 1) * NR
MRPT = MR // 16
BR = 1024
NBLK = NR // BR

_mesh = plsc.VectorSubcoreMesh(core_axis_name="c", subcore_axis_name="s")


@functools.partial(
    pl.kernel,
    out_type=jax.ShapeDtypeStruct((2, NR), jnp.float32),
    mesh=_mesh,
    scratch_types=[
        pltpu.VMEM((ROWS_TILE32, EB), jnp.int32),
        pltpu.VMEM((RPT,), jnp.float32),
        pltpu.VMEM((EB,), jnp.float32),
        pltpu.VMEM_SHARED((NR,), jnp.float32),
    ],
)
def _sc_deg(dst_hbm, out_hbm, idx_d, zbuf, ones, acc):
    c = lax.axis_index("c")
    s = lax.axis_index("s")
    w = c * 16 + s
    pltpu.sync_copy(dst_hbm.at[pl.ds(w * ROWS_TILE32, ROWS_TILE32)], idx_d)

    def _fill(i, _):
        zbuf[pl.ds(i * 16, 16)] = jnp.zeros((16,), jnp.float32)
        return 0
    lax.fori_loop(0, RPT // 16, _fill, 0)

    def _fill1(i, _):
        ones[pl.ds(i * 16, 16)] = jnp.full((16,), 1.0, jnp.float32)
        return 0
    lax.fori_loop(0, EB // 16, _fill1, 0)

    pltpu.sync_copy(zbuf, acc.at[pl.ds(s * RPT, RPT)])
    plsc.subcore_barrier()

    def _scat(j, _):
        pltpu.sync_copy(ones, acc.at[idx_d.at[j]], add=True)
        return 0
    lax.fori_loop(0, ROWS_TILE32, _scat, 0)
    plsc.subcore_barrier()
    pltpu.sync_copy(acc.at[pl.ds(s * RPT, RPT)], out_hbm.at[c, pl.ds(s * RPT, RPT)])


def _make_sc_agg(nch):
    per = nch // 2

    @functools.partial(
        pl.kernel,
        out_type=jax.ShapeDtypeStruct((nch, NR, 128), jnp.float32),
        mesh=_mesh,
        scratch_types=[
            pltpu.VMEM((ROWS_GRP, EB), jnp.int32),
            pltpu.VMEM((ROWS_GRP, EB), jnp.int32),
            pltpu.VMEM((EB, 128), jnp.float32),
            pltpu.VMEM((EB, 128), jnp.float32),
            pltpu.SemaphoreType.DMA,
            pltpu.SemaphoreType.DMA,
            pltpu.VMEM_SHARED((NR, 128), jnp.float32),
        ],
    )
    def _agg(t_hbm, src_hbm, dst_hbm, out_hbm, idx_s, idx_d, bufa, bufb,
             sema, semb, acc):
        c = lax.axis_index("c")
        s = lax.axis_index("s")
        base = s * ROWS_TILE
        row0 = s * RPT
        for k in range(per):
            chunk = per * c + k
            tbl = t_hbm.at[chunk]
            pltpu.sync_copy(tbl.at[pl.ds(row0, RPT)], acc.at[pl.ds(row0, RPT)])
            plsc.subcore_barrier()

            for g in range(ROWS_TILE // ROWS_GRP):
                pltpu.sync_copy(
                    src_hbm.at[pl.ds(base + g * ROWS_GRP, ROWS_GRP)], idx_s)
                pltpu.sync_copy(
                    dst_hbm.at[pl.ds(base + g * ROWS_GRP, ROWS_GRP)], idx_d)
                pltpu.async_copy(tbl.at[idx_s.at[0]], bufa, sema)

                def _pair(i, _):
                    j0 = 2 * i
                    pltpu.async_copy(tbl.at[idx_s.at[j0 + 1]], bufb, semb)
                    pltpu.make_async_copy(tbl.at[idx_s.at[j0]], bufa,
                                          sema).wait()
                    pltpu.sync_copy(bufa, acc.at[idx_d.at[j0]], add=True)

                    @pl.when(i < ROWS_GRP // 2 - 1)
                    def _():
                        pltpu.async_copy(tbl.at[idx_s.at[j0 + 2]], bufa, sema)

                    pltpu.make_async_copy(tbl.at[idx_s.at[j0 + 1]], bufb,
                                          semb).wait()
                    pltpu.sync_copy(bufb, acc.at[idx_d.at[j0 + 1]], add=True)
                    return 0
                lax.fori_loop(0, ROWS_GRP // 2, _pair, 0)
            plsc.subcore_barrier()
            pltpu.sync_copy(acc.at[pl.ds(row0, RPT)],
                            out_hbm.at[chunk, pl.ds(row0, RPT)])
            plsc.subcore_barrier()
    return _agg


_sc_agg2 = _make_sc_agg(2)
_sc_agg4 = _make_sc_agg(4)


@functools.partial(
    pl.kernel,
    out_type=jax.ShapeDtypeStruct((2, MR), jnp.float32),
    mesh=_mesh,
    scratch_types=[
        pltpu.VMEM((ET32,), jnp.int32),
        pltpu.VMEM((ET32,), jnp.int32),
        pltpu.VMEM((NR,), jnp.int32),
        pltpu.VMEM((NR,), jnp.float32),
        pltpu.VMEM((EB,), jnp.int32),
        pltpu.VMEM((EB,), jnp.float32),
        pltpu.VMEM((MRPT,), jnp.float32),
        pltpu.VMEM_SHARED((MR,), jnp.float32),
    ],
    compiler_params=pltpu.CompilerParams(needs_layout_passes=False),
)
def _sc_m(src_hbm, dst_hbm, batch_hbm, dinv_hbm, out_hbm,
          ssrc, sdst, btab, dtab, fx, fv, zbuf, accm):
    c = lax.axis_index("c")
    s = lax.axis_index("s")
    w = c * 16 + s
    pltpu.sync_copy(src_hbm.at[pl.ds(w * ET32, ET32)], ssrc)
    pltpu.sync_copy(dst_hbm.at[pl.ds(w * ET32, ET32)], sdst)
    pltpu.sync_copy(batch_hbm, btab)
    pltpu.sync_copy(dinv_hbm, dtab)

    def _fill(i, _):
        zbuf[pl.ds(i * 16, 16)] = jnp.zeros((16,), jnp.float32)
        return 0
    lax.fori_loop(0, MRPT // 16, _fill, 0)
    pltpu.sync_copy(zbuf, accm.at[pl.ds(s * MRPT, MRPT)])
    plsc.subcore_barrier()

    def _blk(j, _):
        for m in range(EB // 16):
            off = j * EB + m * 16
            d16 = sdst[pl.ds(off, 16)]
            s16 = ssrc[pl.ds(off, 16)]
            b16 = plsc.load_gather(btab, [d16])
            v16 = plsc.load_gather(dtab, [d16])
            fx[pl.ds(m * 16, 16)] = b16 * NR + s16
            fv[pl.ds(m * 16, 16)] = v16
        pltpu.sync_copy(fv, accm.at[fx], add=True)
        return 0
    lax.fori_loop(0, ET32 // EB, _blk, 0)
    plsc.subcore_barrier()
    pltpu.sync_copy(accm.at[pl.ds(s * MRPT, MRPT)],
                    out_hbm.at[c, pl.ds(s * MRPT, MRPT)])


def _dinv_of(deg_ref):
    cnt = deg_ref[0, :] + deg_ref[1, :]
    return lax.rsqrt(cnt + 1.0)


def _tc_dinv(deg2):
    def body(deg_ref, out_ref):
        out_ref[0, 0] = _dinv_of(deg_ref)

    return pl.pallas_call(
        body,
        grid=(NBLK,),
        in_specs=[pl.BlockSpec((2, BR), lambda i: (0, i))],
        out_specs=pl.BlockSpec((1, 1, BR), lambda i: (i, 0, 0)),
        out_shape=jax.ShapeDtypeStruct((NBLK, 1, BR), jnp.float32),
    )(deg2)


def _tc_scale_x(xp, deg2):
    def body(x_ref, deg_ref, out_ref):
        xs = x_ref[...] * _dinv_of(deg_ref)[:, None]
        for cc in range(2):
            out_ref[cc] = xs[:, cc * 128:(cc + 1) * 128]

    return pl.pallas_call(
        body,
        grid=(NBLK,),
        in_specs=[
            pl.BlockSpec((BR, D_IN), lambda i: (i, 0)),
            pl.BlockSpec((2, BR), lambda i: (0, i)),
        ],
        out_specs=pl.BlockSpec((2, BR, 128), lambda i: (0, i, 0)),
        out_shape=jax.ShapeDtypeStruct((2, NR, 128), jnp.float32),
    )(xp, deg2)


def _tc_l1(Sx, deg2, W1, b1, W2):
    def body(s_ref, deg_ref, w1_ref, b1_ref, w2_ref, out_ref):
        ax = jnp.concatenate([s_ref[0], s_ref[1]], axis=1)
        dinv = _dinv_of(deg_ref)
        ax = ax * dinv[:, None]
        h1 = jnp.dot(ax, w1_ref[...], preferred_element_type=jnp.float32)
        u = jnp.maximum(h1 + b1_ref[...], 0.0)
        t = jnp.dot(u, w2_ref[...], preferred_element_type=jnp.float32)
        t = t * dinv[:, None]
        for cc in range(4):
            out_ref[cc] = t[:, cc * 128:(cc + 1) * 128]

    return pl.pallas_call(
        body,
        grid=(NBLK,),
        in_specs=[
            pl.BlockSpec((2, BR, 128), lambda i: (0, i, 0)),
            pl.BlockSpec((2, BR), lambda i: (0, i)),
            pl.BlockSpec((D_IN, D_H), lambda i: (0, 0)),
            pl.BlockSpec((1, D_H), lambda i: (0, 0)),
            pl.BlockSpec((D_H, D_H), lambda i: (0, 0)),
        ],
        out_specs=pl.BlockSpec((4, BR, 128), lambda i: (0, i, 0)),
        out_shape=jax.ShapeDtypeStruct((4, NR, 128), jnp.float32),
    )(Sx, deg2, W1, b1, W2)


def _tc_l3pool(S2, deg2, b2, batch2d, Mp3, W3, b3, Wm1, bm1, Wm2, bm2):
    def body(s_ref, deg_ref, b2_ref, batch_ref, mp_ref, w3_ref, b3_ref,
             wm1_ref, bm1_ref, wm2_ref, bm2_ref, out_ref, musum, cnts):
        i = pl.program_id(0)

        @pl.when(i == 0)
        def _init():
            musum[...] = jnp.zeros((NUM_GRAPHS, D_H), jnp.float32)
            cnts[...] = jnp.zeros((NUM_GRAPHS, 128), jnp.float32)

        h = jnp.concatenate([s_ref[cc] for cc in range(4)], axis=1)
        dinv = _dinv_of(deg_ref)
        u3 = jnp.maximum(h * dinv[:, None] + b2_ref[...], 0.0)
        bvec = batch_ref[0, 0]
        gids = lax.broadcasted_iota(jnp.int32, (NUM_GRAPHS, 1), 0)
        pg = (bvec[None, :] == gids).astype(jnp.float32)
        mblk = mp_ref[0, :NUM_GRAPHS, :] + mp_ref[1, :NUM_GRAPHS, :]
        mfull = (mblk + pg * dinv[None, :]) * dinv[None, :]
        dn = (((1,), (0,)), ((), ()))
        musum[...] += lax.dot_general(mfull, u3, dn,
                                      preferred_element_type=jnp.float32)
        cnts[...] += lax.dot_general(pg, jnp.ones((BR, 128), jnp.float32), dn,
                                     preferred_element_type=jnp.float32)

        @pl.when(i == NBLK - 1)
        def _fin():
            cnt = cnts[:, 0:1]
            zsum = jnp.dot(musum[...], w3_ref[...],
                           preferred_element_type=jnp.float32) + cnt * b3_ref[...]
            z = zsum / jnp.maximum(cnt, 1.0)
            z = jnp.maximum(
                jnp.dot(z, wm1_ref[...], preferred_element_type=jnp.float32)
                + bm1_ref[...], 0.0)
            out_ref[...] = (
                jnp.dot(z, wm2_ref[...], preferred_element_type=jnp.float32)
                + bm2_ref[...])

    return pl.pallas_call(
        body,
        grid=(NBLK,),
        in_specs=[
            pl.BlockSpec((4, BR, 128), lambda i: (0, i, 0)),
            pl.BlockSpec((2, BR), lambda i: (0, i)),
            pl.BlockSpec((1, D_H), lambda i: (0, 0)),
            pl.BlockSpec((1, 1, BR), lambda i: (i, 0, 0)),
            pl.BlockSpec((2, NUM_GRAPHS + 1, BR), lambda i: (0, 0, i)),
            pl.BlockSpec((D_H, D_H), lambda i: (0, 0)),
            pl.BlockSpec((1, D_H), lambda i: (0, 0)),
            pl.BlockSpec((D_H, D_H // 2), lambda i: (0, 0)),
            pl.BlockSpec((1, D_H // 2), lambda i: (0, 0)),
            pl.BlockSpec((D_H // 2, NUM_EXPERTS), lambda i: (0, 0)),
            pl.BlockSpec((1, NUM_EXPERTS), lambda i: (0, 0)),
        ],
        out_specs=pl.BlockSpec((NUM_GRAPHS, NUM_EXPERTS), lambda i: (0, 0)),
        out_shape=jax.ShapeDtypeStruct((NUM_GRAPHS, NUM_EXPERTS), jnp.float32),
        scratch_shapes=[
            pltpu.VMEM((NUM_GRAPHS, D_H), jnp.float32),
            pltpu.VMEM((NUM_GRAPHS, 128), jnp.float32),
        ],
    )(S2, deg2, b2, batch2d, Mp3, W3, b3, Wm1, bm1, Wm2, bm2)


def kernel(x, edge_index, batch, W1, b1, W2, b2, W3, b3, Wm1, bm1, Wm2, bm2):
    src = edge_index[0]
    dst = edge_index[1]
    src2d = jnp.pad(src, (0, EP - E)).reshape(ROWS_ALL, EB)
    dst2d = jnp.pad(dst, (0, EP - E), constant_values=N).reshape(ROWS_ALL, EB)
    xp = jnp.pad(x, ((0, NR - N), (0, 0)))
    batch2d = jnp.pad(batch, (0, NR - N),
                      constant_values=NUM_GRAPHS).reshape(NBLK, 1, BR)

    deg2 = _sc_deg(dst2d)
    dinv_flat = _tc_dinv(deg2).reshape(NR)
    batch_flat = jnp.pad(batch, (0, NR - N), constant_values=NUM_GRAPHS)
    Mp = _sc_m(src2d.reshape(EP), dst2d.reshape(EP), batch_flat, dinv_flat)
    xs = _tc_scale_x(xp, deg2)
    Sx = _sc_agg2(xs, src2d, dst2d)
    t2 = _tc_l1(Sx, deg2, W1, b1.reshape(1, D_H), W2)
    S2 = _sc_agg4(t2, src2d, dst2d)
    return _tc_l3pool(S2, deg2, b2.reshape(1, D_H), batch2d,
                      Mp.reshape(2, NUM_GRAPHS + 1, NR), W3,
                      b3.reshape(1, D_H), Wm1, bm1.reshape(1, D_H // 2),
                      Wm2, bm2.reshape(1, NUM_EXPERTS))
